# baseline (speedup 1.0000x reference)
import functools
import math

import jax
import jax.numpy as jnp
import numpy as np
from jax import lax
from jax.experimental import pallas as pl
from jax.experimental.pallas import tpu as pltpu
from jax.experimental.pallas import tpu_sc as plsc

VOCAB = 100000
D_MODEL = 1024
MAX_LEN = 2048
BATCH = 4
SEQ_LEN = 2048

NUM_CORES = 2
NUM_SUBCORES = 16
NUM_WORKERS = NUM_CORES * NUM_SUBCORES
POS_PER_WORKER = SEQ_LEN // NUM_WORKERS
CHUNK = 16
GROUPS = POS_PER_WORKER // CHUNK
STEPS = GROUPS * BATCH
NBUF = 4
LANES = 16
SLICES_PER_ROW = D_MODEL // LANES
SLICES_PER_STEP = CHUNK * SLICES_PER_ROW
SCALE = math.sqrt(D_MODEL)


def _make_pe(max_len, d_model):
    pe = np.zeros((max_len, d_model), dtype=np.float32)
    position = np.arange(0, max_len, dtype=np.float32)[:, None]
    div_term = np.exp(
        np.arange(0, d_model, 2, dtype=np.float32) * -(math.log(10000.0) / d_model)
    )
    pe[:, 0::2] = np.sin(position * div_term)
    pe[:, 1::2] = np.cos(position * div_term)
    return pe


_PE = _make_pe(MAX_LEN, D_MODEL)


def _emb_kernel(table, idx_hbm, pe_hbm, out, *refs):
    rows = refs[0:NBUF]
    pe_a, pe_b = refs[NBUF : NBUF + 2]
    idxs = refs[NBUF + 2 : NBUF + 2 + STEPS]
    sems = refs[NBUF + 2 + STEPS :]
    gsems = sems[0:NBUF]
    ssems = sems[NBUF : 2 * NBUF]
    psem_a, psem_b = sems[2 * NBUF : 2 * NBUF + 2]

    wid = lax.axis_index("s") * NUM_CORES + lax.axis_index("c")
    s0 = wid * POS_PER_WORKER

    pes = [(pe_a, psem_a), (pe_b, psem_b)]

    def issue_pe(pg):
        buf, sem = pes[pg % 2]
        return pltpu.async_copy(pe_hbm.at[pl.ds(s0 + pg * CHUNK, CHUNK)], buf, sem)

    def issue_gather(i):
        return pltpu.async_copy(table.at[idxs[i]], rows[i % NBUF], gsems[i % NBUF])

    def compute(i):
        pe_buf = pes[(i >> 2) % 2][0]
        buf = rows[i % NBUF]

        @plsc.parallel_loop(0, SLICES_PER_STEP, unroll=4)
        def _(s):
            r = lax.shift_right_logical(s, 6)
            col = pl.multiple_of(lax.bitwise_and(s, 63) * LANES, LANES)
            sl = pl.ds(col, LANES)
            buf[r, sl] = buf[r, sl] * SCALE + pe_buf[r, sl]

    def issue_store(i):
        pg, b = i >> 2, i & 3
        ooff = b * SEQ_LEN + s0 + pg * CHUNK
        return pltpu.async_copy(rows[i % NBUF], out.at[pl.ds(ooff, CHUNK)], ssems[i % NBUF])

    def wait_gather(i):
        pltpu.make_async_copy(
            pe_hbm.at[pl.ds(0, CHUNK)], rows[i % NBUF], gsems[i % NBUF]
        ).wait()

    def wait_store(i):
        pltpu.make_async_copy(
            rows[i % NBUF], out.at[pl.ds(0, CHUNK)], ssems[i % NBUF]
        ).wait()

    issue_pe(0)
    for i in range(STEPS):
        pg, b = i >> 2, i & 3
        pltpu.sync_copy(idx_hbm.at[pl.ds(b * SEQ_LEN + s0 + pg * CHUNK, CHUNK)], idxs[i])
    issue_gather(0)
    issue_gather(1)

    for i in range(STEPS):
        pg, b = i >> 2, i & 3
        if b == 0 and pg + 1 < GROUPS:
            issue_pe(pg + 1)
        if i + 2 < STEPS:
            if i >= 2:
                wait_store(i - 2)
            issue_gather(i + 2)
        wait_gather(i)
        if b == 0:
            buf, sem = pes[pg % 2]
            pltpu.make_async_copy(pe_hbm.at[pl.ds(0, CHUNK)], buf, sem).wait()
        compute(i)
        issue_store(i)

    for i in range(STEPS - NBUF, STEPS):
        wait_store(i)


@jax.jit
def _run(x_flat, emb_table, pe):
    mesh = plsc.VectorSubcoreMesh(core_axis_name="c", subcore_axis_name="s")
    k = functools.partial(
        pl.kernel,
        mesh=mesh,
        out_type=jax.ShapeDtypeStruct((BATCH * SEQ_LEN, D_MODEL), jnp.float32),
        scratch_types=(
            [pltpu.VMEM((CHUNK, D_MODEL), jnp.float32) for _ in range(NBUF)]
            + [
                pltpu.VMEM((CHUNK, D_MODEL), jnp.float32),
                pltpu.VMEM((CHUNK, D_MODEL), jnp.float32),
            ]
            + [pltpu.VMEM((CHUNK,), jnp.int32) for _ in range(STEPS)]
            + [pltpu.SemaphoreType.DMA for _ in range(2 * NBUF + 2)]
        ),
    )(_emb_kernel)
    return k(emb_table, x_flat, pe)


def kernel(x, emb_table):
    x_flat = x.reshape(BATCH * SEQ_LEN).astype(jnp.int32)
    out = _run(x_flat, emb_table, jnp.asarray(_PE))
    return out.reshape(BATCH, SEQ_LEN, D_MODEL)


# speedup vs baseline: 1.0765x; 1.0765x over previous
import functools
import math

import jax
import jax.numpy as jnp
import numpy as np
from jax import lax
from jax.experimental import pallas as pl
from jax.experimental.pallas import tpu as pltpu
from jax.experimental.pallas import tpu_sc as plsc

VOCAB = 100000
D_MODEL = 1024
MAX_LEN = 2048
BATCH = 4
SEQ_LEN = 2048

NUM_CORES = 2
NUM_SUBCORES = 16
NUM_WORKERS = NUM_CORES * NUM_SUBCORES
POS_PER_WORKER = SEQ_LEN // NUM_WORKERS
CHUNK = 16
HALF = CHUNK // 2
GROUPS = POS_PER_WORKER // CHUNK
STEPS = GROUPS * BATCH
NBUF = 5
LOOKAHEAD = 3
LANES = 16
SLICES_PER_ROW = D_MODEL // LANES
SLICES_PER_HALF = HALF * SLICES_PER_ROW
SCALE = math.sqrt(D_MODEL)


def _make_pe(max_len, d_model):
    pe = np.zeros((max_len, d_model), dtype=np.float32)
    position = np.arange(0, max_len, dtype=np.float32)[:, None]
    div_term = np.exp(
        np.arange(0, d_model, 2, dtype=np.float32) * -(math.log(10000.0) / d_model)
    )
    pe[:, 0::2] = np.sin(position * div_term)
    pe[:, 1::2] = np.cos(position * div_term)
    return pe


_PE = _make_pe(MAX_LEN, D_MODEL)


def _emb_kernel(table, idx_hbm, pe_hbm, out, *refs):
    rows = refs[0:NBUF]
    pe_a, pe_b, idx_v = refs[NBUF : NBUF + 3]
    sems = refs[NBUF + 3 :]
    gsems = sems[0:NBUF]
    ssems = sems[NBUF : 2 * NBUF]
    psem_a, psem_b = sems[2 * NBUF : 2 * NBUF + 2]

    wid = lax.axis_index("s") * NUM_CORES + lax.axis_index("c")
    s0 = wid * POS_PER_WORKER

    pes = [(pe_a, psem_a), (pe_b, psem_b)]

    def issue_pe(pg):
        buf, sem = pes[pg % 2]
        return pltpu.async_copy(pe_hbm.at[pl.ds(s0 + pg * CHUNK, CHUNK)], buf, sem)

    def issue_gather(i):
        pg, b = i >> 2, i & 3
        ioff = b * POS_PER_WORKER + pg * CHUNK
        return pltpu.async_copy(
            table.at[idx_v.at[pl.ds(ioff, CHUNK)]], rows[i % NBUF], gsems[i % NBUF]
        )

    def compute_half(i, h):
        pe_buf = pes[(i >> 2) % 2][0]
        buf = rows[i % NBUF]
        rbase = h * HALF

        @plsc.parallel_loop(0, SLICES_PER_HALF, unroll=4)
        def _(s):
            r = rbase + lax.shift_right_logical(s, 6)
            col = pl.multiple_of(lax.bitwise_and(s, 63) * LANES, LANES)
            sl = pl.ds(col, LANES)
            buf[r, sl] = buf[r, sl] * SCALE + pe_buf[r, sl]

    def issue_store_half(i, h):
        pg, b = i >> 2, i & 3
        ooff = b * SEQ_LEN + s0 + pg * CHUNK + h * HALF
        return pltpu.async_copy(
            rows[i % NBUF].at[pl.ds(h * HALF, HALF)],
            out.at[pl.ds(ooff, HALF)],
            ssems[i % NBUF],
        )

    def wait_gather(i):
        pltpu.make_async_copy(
            out.at[pl.ds(0, CHUNK)], rows[i % NBUF], gsems[i % NBUF]
        ).wait()

    def wait_store(i):
        for _ in range(2):
            pltpu.make_async_copy(
                rows[i % NBUF].at[pl.ds(0, HALF)],
                out.at[pl.ds(0, HALF)],
                ssems[i % NBUF],
            ).wait()

    issue_pe(0)
    for b in range(BATCH):
        pltpu.sync_copy(
            idx_hbm.at[pl.ds(b * SEQ_LEN + s0, POS_PER_WORKER)],
            idx_v.at[pl.ds(b * POS_PER_WORKER, POS_PER_WORKER)],
        )
    for i in range(LOOKAHEAD):
        issue_gather(i)

    for i in range(STEPS):
        pg, b = i >> 2, i & 3
        if b == 0 and pg + 1 < GROUPS:
            issue_pe(pg + 1)
        if i + LOOKAHEAD < STEPS:
            if i + LOOKAHEAD >= NBUF:
                wait_store(i + LOOKAHEAD - NBUF)
            issue_gather(i + LOOKAHEAD)
        wait_gather(i)
        if b == 0:
            buf, sem = pes[pg % 2]
            pltpu.make_async_copy(pe_hbm.at[pl.ds(0, CHUNK)], buf, sem).wait()
        compute_half(i, 0)
        issue_store_half(i, 0)
        compute_half(i, 1)
        issue_store_half(i, 1)

    for i in range(STEPS - NBUF, STEPS):
        wait_store(i)


@jax.jit
def _run(x_flat, emb_table, pe):
    mesh = plsc.VectorSubcoreMesh(core_axis_name="c", subcore_axis_name="s")
    k = functools.partial(
        pl.kernel,
        mesh=mesh,
        out_type=jax.ShapeDtypeStruct((BATCH * SEQ_LEN, D_MODEL), jnp.float32),
        scratch_types=(
            [pltpu.VMEM((CHUNK, D_MODEL), jnp.float32) for _ in range(NBUF)]
            + [
                pltpu.VMEM((CHUNK, D_MODEL), jnp.float32),
                pltpu.VMEM((CHUNK, D_MODEL), jnp.float32),
                pltpu.VMEM((BATCH * POS_PER_WORKER,), jnp.int32),
            ]
            + [pltpu.SemaphoreType.DMA for _ in range(2 * NBUF + 2)]
        ),
    )(_emb_kernel)
    return k(emb_table, x_flat, pe)


def kernel(x, emb_table):
    x_flat = x.reshape(BATCH * SEQ_LEN).astype(jnp.int32)
    out = _run(x_flat, emb_table, jnp.asarray(_PE))
    return out.reshape(BATCH, SEQ_LEN, D_MODEL)
